# baseline (speedup 1.0000x reference)
import numpy as np
import jax
import jax.numpy as jnp
from jax.experimental import pallas as pl
from jax.experimental.pallas import tpu as pltpu

_K = 8192
_D = 64
_BLK = 8192
_NSTEP = _K // _BLK


def _threefry_bits_transposed() -> np.ndarray:
    n = _K * _D
    with np.errstate(over="ignore"):
        x1 = np.arange(n, dtype=np.uint32)
        x0 = np.zeros(n, dtype=np.uint32)
        k0, k1 = np.uint32(0), np.uint32(42)
        ks = [k0, k1, np.uint32(int(k0) ^ int(k1) ^ 0x1BD11BDA)]
        x0 = x0 + ks[0]
        x1 = x1 + ks[1]
        rot = ((13, 15, 26, 6), (17, 29, 16, 24))
        for g in range(5):
            for r in rot[g % 2]:
                x0 = (x0 + x1).astype(np.uint32)
                x1 = ((x1 << np.uint32(r)) | (x1 >> np.uint32(32 - r))).astype(np.uint32)
                x1 = x1 ^ x0
            x0 = (x0 + ks[(g + 1) % 3]).astype(np.uint32)
            x1 = (x1 + ks[(g + 2) % 3] + np.uint32(g + 1)).astype(np.uint32)
        bits = x0 ^ x1
    return np.ascontiguousarray(bits.reshape(_K, _D).T)


_BITS_T = _threefry_bits_transposed()


def _pure_index_body(wt_ref, bits_ref, og_ref, idx_ref, m_acc, i_acc):
    j = pl.program_id(0)
    wt = wt_ref[...]
    bits = bits_ref[...]
    fbits = (bits >> jnp.uint32(9)) | jnp.uint32(0x3F800000)
    f = jax.lax.bitcast_convert_type(fbits, jnp.float32) - jnp.float32(1.0)
    u = f + jnp.float32(1e-20)
    g = -jnp.log(-jnp.log(u))
    z = wt + g
    zmax = jnp.max(z, axis=0, keepdims=True)
    e = jnp.exp(z - zmax)
    s = jnp.sum(e, axis=0, keepdims=True)
    y = e / s
    m = jnp.float32(1.0) / s
    c = jax.lax.broadcasted_iota(jnp.int32, (_D, _BLK), 0)
    idx = jnp.min(jnp.where(y == m, c, _D), axis=0, keepdims=True)
    m_acc[:, pl.ds(j * _BLK, _BLK)] = (jnp.float32(1.0) - m) + m
    i_acc[:, pl.ds(j * _BLK, _BLK)] = idx

    @pl.when(j == _NSTEP - 1)
    def _assemble():
        og_ref[pl.ds(0, 1)] = jnp.full((1,), 1.0, jnp.float32)
        og_ref[pl.ds(1, _K)] = m_acc[0, :]
        og_ref[pl.ds(_K + 1, 1)] = jnp.full((1,), 1.0, jnp.float32)
        idx_ref[pl.ds(0, 1)] = jnp.full((1,), _D, jnp.int32)
        idx_ref[pl.ds(1, _K)] = i_acc[0, :]
        idx_ref[pl.ds(_K + 1, 1)] = jnp.full((1,), _D + 1, jnp.int32)


def kernel(image, W, step):
    del image, step
    bits = jnp.asarray(_BITS_T)
    og, idx = pl.pallas_call(
        _pure_index_body,
        grid=(_NSTEP,),
        in_specs=[
            pl.BlockSpec((_D, _BLK), lambda j: (0, j)),
            pl.BlockSpec((_D, _BLK), lambda j: (0, j)),
        ],
        out_specs=[
            pl.BlockSpec((_K + 2,), lambda j: (0,)),
            pl.BlockSpec((_K + 2,), lambda j: (0,)),
        ],
        out_shape=[
            jax.ShapeDtypeStruct((_K + 2,), jnp.float32),
            jax.ShapeDtypeStruct((_K + 2,), jnp.int32),
        ],
        scratch_shapes=[
            pltpu.VMEM((1, _K), jnp.float32),
            pltpu.VMEM((1, _K), jnp.int32),
        ],
    )(W.T, bits)
    return (og, idx)


# speedup vs baseline: 1.0990x; 1.0990x over previous
import numpy as np
import jax
import jax.numpy as jnp
from jax.experimental import pallas as pl
from jax.experimental.pallas import tpu as pltpu

_K = 8192
_D = 64
_BLK = 4096
_NSTEP = _K // _BLK


def _threefry_bits_transposed() -> np.ndarray:
    n = _K * _D
    with np.errstate(over="ignore"):
        x1 = np.arange(n, dtype=np.uint32)
        x0 = np.zeros(n, dtype=np.uint32)
        k0, k1 = np.uint32(0), np.uint32(42)
        ks = [k0, k1, np.uint32(int(k0) ^ int(k1) ^ 0x1BD11BDA)]
        x0 = x0 + ks[0]
        x1 = x1 + ks[1]
        rot = ((13, 15, 26, 6), (17, 29, 16, 24))
        for g in range(5):
            for r in rot[g % 2]:
                x0 = (x0 + x1).astype(np.uint32)
                x1 = ((x1 << np.uint32(r)) | (x1 >> np.uint32(32 - r))).astype(np.uint32)
                x1 = x1 ^ x0
            x0 = (x0 + ks[(g + 1) % 3]).astype(np.uint32)
            x1 = (x1 + ks[(g + 2) % 3] + np.uint32(g + 1)).astype(np.uint32)
        bits = x0 ^ x1
    return np.ascontiguousarray(bits.reshape(_K, _D).T)


_BITS_T = _threefry_bits_transposed()


def _pure_index_body(wt_ref, bits_ref, og_ref, idx_ref, m_acc, i_acc):
    j = pl.program_id(0)
    wt = wt_ref[...]
    bits = bits_ref[...]
    fbits = (bits >> jnp.uint32(9)) | jnp.uint32(0x3F800000)
    f = jax.lax.bitcast_convert_type(fbits, jnp.float32) - jnp.float32(1.0)
    u = f + jnp.float32(1e-20)
    g = -jnp.log(-jnp.log(u))
    z = wt + g
    zmax = jnp.max(z, axis=0, keepdims=True)
    e = jnp.exp(z - zmax)
    s = jnp.sum(e, axis=0, keepdims=True)
    y = e / s
    m = jnp.float32(1.0) / s
    c = jax.lax.broadcasted_iota(jnp.int32, (_D, _BLK), 0)
    idx = jnp.min(jnp.where(y == m, c, _D), axis=0, keepdims=True)
    m_acc[:, pl.ds(j * _BLK, _BLK)] = (jnp.float32(1.0) - m) + m
    i_acc[:, pl.ds(j * _BLK, _BLK)] = idx

    @pl.when(j == _NSTEP - 1)
    def _assemble():
        og_ref[pl.ds(0, 1)] = jnp.full((1,), 1.0, jnp.float32)
        og_ref[pl.ds(1, _K)] = m_acc[0, :]
        og_ref[pl.ds(_K + 1, 1)] = jnp.full((1,), 1.0, jnp.float32)
        idx_ref[pl.ds(0, 1)] = jnp.full((1,), _D, jnp.int32)
        idx_ref[pl.ds(1, _K)] = i_acc[0, :]
        idx_ref[pl.ds(_K + 1, 1)] = jnp.full((1,), _D + 1, jnp.int32)


def kernel(image, W, step):
    del image, step
    bits = jnp.asarray(_BITS_T)
    og, idx = pl.pallas_call(
        _pure_index_body,
        grid=(_NSTEP,),
        in_specs=[
            pl.BlockSpec((_D, _BLK), lambda j: (0, j)),
            pl.BlockSpec((_D, _BLK), lambda j: (0, j)),
        ],
        out_specs=[
            pl.BlockSpec((_K + 2,), lambda j: (0,)),
            pl.BlockSpec((_K + 2,), lambda j: (0,)),
        ],
        out_shape=[
            jax.ShapeDtypeStruct((_K + 2,), jnp.float32),
            jax.ShapeDtypeStruct((_K + 2,), jnp.int32),
        ],
        scratch_shapes=[
            pltpu.VMEM((1, _K), jnp.float32),
            pltpu.VMEM((1, _K), jnp.int32),
        ],
    )(W.T, bits)
    return (og, idx)
